# baseline (speedup 1.0000x reference)
import functools
import jax
import jax.numpy as jnp
from jax import lax
from jax.experimental import pallas as pl
from jax.experimental.pallas import tpu as pltpu
from jax.experimental.pallas import tpu_sc as plsc

B = 16384
F = 64
NW = 32
ROWS = B // NW
CH = 64
NCH = ROWS // CH

_mesh = plsc.VectorSubcoreMesh(core_axis_name="c", subcore_axis_name="s")


@functools.partial(
    pl.kernel,
    mesh=_mesh,
    out_type=jax.ShapeDtypeStruct((16,), jnp.float32),
    scratch_types=[
        pltpu.VMEM((2, CH, F), jnp.float32),
        pltpu.VMEM((2, CH, F), jnp.float32),
        pltpu.VMEM((16,), jnp.float32),
        pltpu.SemaphoreType.DMA((2,)),
        pltpu.SemaphoreType.DMA((2,)),
    ],
)
def _sc_probe(x_hbm, t_hbm, out_hbm, xb, tb, ov, sx, st):
    c = lax.axis_index("c")
    s = lax.axis_index("s")
    wid = s * 2 + c
    base = wid * ROWS

    def start(i):
        slot = i % 2
        return (
            pltpu.async_copy(
                x_hbm.at[pl.ds(base + i * CH, CH), :], xb.at[slot], sx.at[slot]
            ),
            pltpu.async_copy(
                t_hbm.at[pl.ds(base + i * CH, CH), :], tb.at[slot], st.at[slot]
            ),
        )

    pend = {0: start(0)}
    for i in range(NCH):
        if i + 1 < NCH:
            pend[i + 1] = start(i + 1)
        a, b = pend.pop(i)
        a.wait()
        b.wait()

    @pl.when(wid == 0)
    def _():
        ov[...] = xb[0, 0, pl.ds(0, 16)] + tb[0, 0, pl.ds(0, 16)]
        pltpu.sync_copy(ov, out_hbm)


def kernel(input, target):
    res = _sc_probe(input, target)
    return res[0]


# speedup vs baseline: 1.6916x; 1.6916x over previous
import jax
import jax.numpy as jnp
from jax.experimental import pallas as pl
from jax.experimental.pallas import tpu as pltpu

B = 16384
F = 64
K = int(0.3 * B)
BLK = 8192
GRID = B // BLK


def _body(x_ref, t_ref, out_ref, loss_ref):
    i = pl.program_id(0)
    d = x_ref[...] - t_ref[...]
    part = jnp.sum(d * d, axis=1) * (1.0 / F)
    loss_ref[pl.ds(i * (BLK // 128), BLK // 128), :] = part.reshape(BLK // 128, 128)

    @pl.when(i == GRID - 1)
    def _():
        loss = loss_ref[...]
        keys = jax.lax.bitcast_convert_type(loss, jnp.int32)

        t = jnp.int32(0)
        for g in range(8):
            sh = 28 - 4 * g
            chosen = jnp.int32(0)
            for n in range(1, 8 if g == 0 else 16):
                cand = t | (n << sh)
                cnt = jnp.sum((keys >= cand).astype(jnp.int32))
                chosen = chosen + (cnt >= K).astype(jnp.int32)
            t = t | (chosen << sh)

        gt = keys > t
        c_gt = jnp.sum(gt.astype(jnp.int32))
        s_gt = jnp.sum(jnp.where(gt, loss, 0.0))
        tf = jax.lax.bitcast_convert_type(t, jnp.float32)
        out_ref[0] = (s_gt + (K - c_gt).astype(jnp.float32) * tf) * (1.0 / K)


def kernel(input, target):
    res = pl.pallas_call(
        _body,
        grid=(GRID,),
        in_specs=[
            pl.BlockSpec((BLK, F), lambda i: (i, 0)),
            pl.BlockSpec((BLK, F), lambda i: (i, 0)),
        ],
        out_specs=pl.BlockSpec(memory_space=pltpu.SMEM),
        out_shape=jax.ShapeDtypeStruct((1,), jnp.float32),
        scratch_shapes=[pltpu.VMEM((128, 128), jnp.float32)],
    )(input, target)
    return res[0]
